# baseline (speedup 1.0000x reference)
import functools

import jax
import jax.numpy as jnp
from jax import lax
from jax.experimental import pallas as pl
from jax.experimental.pallas import tpu as pltpu

H = 16
D = 64
BD = 1024
M = 1024
IN = 1024
Z = 64


def _head_body(x_ref, wq_ref, bq_ref, p_ref, w_ref, out_ref):
    q = lax.dot_general(
        x_ref[...], wq_ref[...], (((1,), (1,)), ((), ())),
        preferred_element_type=jnp.float32,
    ) + bq_ref[0]
    qn = q * lax.rsqrt(jnp.sum(q * q))

    p = p_ref[0]
    s = lax.dot_general(qn, p, (((1,), (1,)), ((), ())),
                        preferred_element_type=jnp.float32)

    smax = jnp.max(s)
    hi0 = smax + jnp.maximum(jnp.abs(smax), 1.0) * 1e-6
    lo0 = jnp.min(s)

    def bis(_, carry):
        lo, hi = carry
        mid = 0.5 * (lo + hi)
        ge = jnp.sum((s >= mid).astype(jnp.int32)) >= Z
        return (jnp.where(ge, mid, lo), jnp.where(ge, hi, mid))

    lo, _ = lax.fori_loop(0, 48, bis, (lo0, hi0))
    mask = (s >= lo).astype(jnp.float32)

    w = w_ref[0]
    attn = lax.dot_general(mask, w, (((1,), (1,)), ((), ())),
                           preferred_element_type=jnp.float32)

    amx = jnp.max(attn)
    idx = lax.broadcasted_iota(jnp.int32, (1, M), 1)
    top = jnp.min(jnp.where(attn == amx, idx, M))
    onehot = (idx == top).astype(jnp.float32)

    row = lax.dot_general(onehot, w, (((1,), (0,)), ((), ())),
                          preferred_element_type=jnp.float32)
    o = lax.dot_general(row, p, (((1,), (0,)), ((), ())),
                        preferred_element_type=jnp.float32)
    out_ref[0] = o * (8.0 * lax.rsqrt(jnp.sum(o * o)))


@jax.jit
def _fused(x2, W_q, b_q2, bin_proj, weight_matrix):
    out = pl.pallas_call(
        _head_body,
        grid=(H,),
        in_specs=[
            pl.BlockSpec((1, IN), lambda h: (0, 0)),
            pl.BlockSpec((D, IN), lambda h: (h, 0)),
            pl.BlockSpec((1, 1, D), lambda h: (h, 0, 0)),
            pl.BlockSpec((1, BD, D), lambda h: (h, 0, 0)),
            pl.BlockSpec((1, M, BD), lambda h: (h, 0, 0)),
        ],
        out_specs=pl.BlockSpec((1, 1, D), lambda h: (h, 0, 0)),
        out_shape=jax.ShapeDtypeStruct((H, 1, D), jnp.float32),
        compiler_params=pltpu.CompilerParams(
            dimension_semantics=("arbitrary",),
        ),
    )(x2, W_q, b_q2, bin_proj, weight_matrix)
    return out


def kernel(x, W_q, b_q, bin_proj, weight_matrix):
    out = _fused(x.reshape(1, IN), W_q, b_q.reshape(H, 1, D), bin_proj,
                 weight_matrix)
    return out.reshape(H * D)


# speedup vs baseline: 2.6277x; 2.6277x over previous
import jax
import jax.numpy as jnp
from jax import lax
from jax.experimental import pallas as pl
from jax.experimental.pallas import tpu as pltpu

H = 16
D = 64
BD = 1024
M = 1024
IN = 1024
Z = 64


def _body(x_ref, wq_ref, bq_ref, p_ref, w_ref, out_ref, mask_ref):
    h = pl.program_id(0)

    @pl.when(h == 0)
    def _setup():
        for i in range(H):
            q = lax.dot_general(
                x_ref[...], wq_ref[i * D:(i + 1) * D, :],
                (((1,), (1,)), ((), ())),
                preferred_element_type=jnp.float32,
            ) + bq_ref[i:i + 1, :]
            qn = q * lax.rsqrt(jnp.sum(q * q))
            s_i = lax.dot_general(qn, p_ref[i], (((1,), (1,)), ((), ())),
                                  preferred_element_type=jnp.float32)
            mask_ref[i:i + 1, :] = s_i
        s = mask_ref[...]

        smax = jnp.max(s, axis=1, keepdims=True)
        hi0 = smax + jnp.maximum(jnp.abs(smax), 1.0) * 1e-6
        lo0 = jnp.min(s, axis=1, keepdims=True)

        def bis(_, carry):
            lo, hi = carry
            mid = 0.5 * (lo + hi)
            cnt = jnp.sum((s >= mid).astype(jnp.int32), axis=1,
                          keepdims=True)
            ge = cnt >= Z
            return (jnp.where(ge, mid, lo), jnp.where(ge, hi, mid))

        lo, _ = lax.fori_loop(0, 48, bis, (lo0, hi0))
        mask_ref[...] = (s >= lo).astype(jnp.float32)

    mrow = mask_ref[pl.ds(h, 1), :]
    w = w_ref[0]
    attn = lax.dot_general(mrow, w, (((1,), (1,)), ((), ())),
                           preferred_element_type=jnp.float32)

    amx = jnp.max(attn)
    idx = lax.broadcasted_iota(jnp.int32, (1, M), 1)
    top = jnp.min(jnp.where(attn == amx, idx, M))
    onehot = (idx == top).astype(jnp.float32)

    row = lax.dot_general(onehot, w, (((1,), (0,)), ((), ())),
                          preferred_element_type=jnp.float32)
    p = p_ref[pl.ds(h, 1)][0]
    o = lax.dot_general(row, p, (((1,), (0,)), ((), ())),
                        preferred_element_type=jnp.float32)
    out_ref[0] = o * (8.0 * lax.rsqrt(jnp.sum(o * o)))


@jax.jit
def _fused(x2, W_q, b_q2, bin_proj, weight_matrix):
    out = pl.pallas_call(
        _body,
        grid=(H,),
        in_specs=[
            pl.BlockSpec((1, IN), lambda h: (0, 0)),
            pl.BlockSpec((IN, IN), lambda h: (0, 0)),
            pl.BlockSpec((H, D), lambda h: (0, 0)),
            pl.BlockSpec((H, BD, D), lambda h: (0, 0, 0)),
            pl.BlockSpec((1, M, BD), lambda h: (h, 0, 0)),
        ],
        out_specs=pl.BlockSpec((1, 1, D), lambda h: (h, 0, 0)),
        out_shape=jax.ShapeDtypeStruct((H, 1, D), jnp.float32),
        scratch_shapes=[pltpu.VMEM((H, BD), jnp.float32)],
        compiler_params=pltpu.CompilerParams(
            dimension_semantics=("arbitrary",),
        ),
    )(x2, W_q, b_q2, bin_proj, weight_matrix)
    return out


def kernel(x, W_q, b_q, bin_proj, weight_matrix):
    out = _fused(x.reshape(1, IN), W_q, b_q.reshape(H, D), bin_proj,
                 weight_matrix)
    return out.reshape(H * D)


# speedup vs baseline: 2.7563x; 1.0489x over previous
import jax
import jax.numpy as jnp
from jax import lax
from jax.experimental import pallas as pl
from jax.experimental.pallas import tpu as pltpu

H = 16
D = 64
BD = 1024
M = 1024
IN = 1024
Z = 64


def _body(x_ref, wq_ref, bq_ref, p_ref, w_ref, out_ref, mask_ref):
    h = pl.program_id(0)

    @pl.when(h == 0)
    def _setup():
        for i in range(H):
            q = lax.dot_general(
                x_ref[...], wq_ref[i * D:(i + 1) * D, :],
                (((1,), (1,)), ((), ())),
                preferred_element_type=jnp.float32,
            ) + bq_ref[i:i + 1, :]
            qn = q * lax.rsqrt(jnp.sum(q * q))
            s_i = lax.dot_general(qn, p_ref[i], (((1,), (1,)), ((), ())),
                                  preferred_element_type=jnp.float32)
            mask_ref[i:i + 1, :] = s_i
        s = mask_ref[...]

        smax = jnp.max(s, axis=1, keepdims=True)
        hi0 = smax + jnp.maximum(jnp.abs(smax), 1.0) * 1e-6
        lo0 = jnp.min(s, axis=1, keepdims=True)

        def bis(_, carry):
            lo, hi = carry
            mid = 0.5 * (lo + hi)
            cnt = jnp.sum((s >= mid).astype(jnp.int32), axis=1,
                          keepdims=True)
            ge = cnt >= Z
            return (jnp.where(ge, mid, lo), jnp.where(ge, hi, mid))

        lo, _ = lax.fori_loop(0, 48, bis, (lo0, hi0))
        mask_ref[...] = (s >= lo).astype(jnp.float32)

    mrow = mask_ref[pl.ds(h, 1), :]
    w = w_ref[0]
    attn = lax.dot_general(mrow, w, (((1,), (1,)), ((), ())),
                           preferred_element_type=jnp.float32)

    amx = jnp.max(attn)
    idx = lax.broadcasted_iota(jnp.int32, (1, M), 1)
    top = jnp.min(jnp.where(attn == amx, idx, M))

    row = w_ref[0, pl.ds(top, 1), :]
    p = p_ref[pl.ds(h, 1)][0]
    o = lax.dot_general(row, p, (((1,), (0,)), ((), ())),
                        preferred_element_type=jnp.float32)
    out_ref[0] = o * (8.0 * lax.rsqrt(jnp.sum(o * o)))


@jax.jit
def _fused(x2, W_q, b_q2, bin_proj, weight_matrix):
    out = pl.pallas_call(
        _body,
        grid=(H,),
        in_specs=[
            pl.BlockSpec((1, IN), lambda h: (0, 0)),
            pl.BlockSpec((IN, IN), lambda h: (0, 0)),
            pl.BlockSpec((H, D), lambda h: (0, 0)),
            pl.BlockSpec((H, BD, D), lambda h: (0, 0, 0)),
            pl.BlockSpec((1, M, BD), lambda h: (h, 0, 0)),
        ],
        out_specs=pl.BlockSpec((1, 1, D), lambda h: (h, 0, 0)),
        out_shape=jax.ShapeDtypeStruct((H, 1, D), jnp.float32),
        scratch_shapes=[pltpu.VMEM((H, BD), jnp.float32)],
        compiler_params=pltpu.CompilerParams(
            dimension_semantics=("arbitrary",),
        ),
    )(x2, W_q, b_q2, bin_proj, weight_matrix)
    return out


def kernel(x, W_q, b_q, bin_proj, weight_matrix):
    out = _fused(x.reshape(1, IN), W_q, b_q.reshape(H, D), bin_proj,
                 weight_matrix)
    return out.reshape(H * D)


# speedup vs baseline: 2.9909x; 1.0851x over previous
import jax
import jax.numpy as jnp
from jax import lax
from jax.experimental import pallas as pl
from jax.experimental.pallas import tpu as pltpu

H = 16
D = 64
BD = 1024
M = 1024
IN = 1024
Z = 64


def _body(x_ref, wq_ref, bq_ref, p_ref, w_ref, out_ref, mask_ref):
    h = pl.program_id(0)

    @pl.when(h == 0)
    def _setup():
        for i in range(H):
            q = lax.dot_general(
                x_ref[...], wq_ref[i * D:(i + 1) * D, :],
                (((1,), (1,)), ((), ())),
                preferred_element_type=jnp.float32,
            ) + bq_ref[i:i + 1, :]
            qn = q * lax.rsqrt(jnp.sum(q * q))
            s_i = lax.dot_general(qn, p_ref[i], (((1,), (1,)), ((), ())),
                                  preferred_element_type=jnp.float32)
            mask_ref[i:i + 1, :] = s_i
        s = mask_ref[...]

        smax = jnp.max(s, axis=1, keepdims=True)
        hi0 = smax + jnp.maximum(jnp.abs(smax), 1.0) * 1e-6
        lo0 = jnp.min(s, axis=1, keepdims=True)

        def bis(_, carry):
            lo, hi = carry
            mid = 0.5 * (lo + hi)
            cnt = jnp.sum((s >= mid).astype(jnp.int32), axis=1,
                          keepdims=True)
            ge = cnt >= Z
            return (jnp.where(ge, mid, lo), jnp.where(ge, hi, mid))

        lo, _ = lax.fori_loop(0, 48, bis, (lo0, hi0))
        mask_ref[...] = (s >= lo).astype(jnp.float32)

    mrow = mask_ref[pl.ds(h, 1), :]
    mb = jnp.broadcast_to(mrow, (8, BD))
    macc = jnp.full((8, 1), -jnp.inf, jnp.float32)
    midx = jnp.zeros((8, 1), jnp.int32)
    for i in range(M // 8):
        wt = w_ref[0, 8 * i:8 * i + 8, :]
        part = jnp.sum(wt * mb, axis=1, keepdims=True)
        upd = part > macc
        macc = jnp.where(upd, part, macc)
        midx = jnp.where(upd, i, midx)

    amx = jnp.max(macc)
    rows = midx * 8 + lax.broadcasted_iota(jnp.int32, (8, 1), 0)
    top = jnp.min(jnp.where(macc == amx, rows, M))

    row = w_ref[0, pl.ds(top, 1), :]
    p = p_ref[pl.ds(h, 1)][0]
    o = lax.dot_general(row, p, (((1,), (0,)), ((), ())),
                        preferred_element_type=jnp.float32)
    out_ref[0] = o * (8.0 * lax.rsqrt(jnp.sum(o * o)))


@jax.jit
def _fused(x2, W_q, b_q2, bin_proj, weight_matrix):
    out = pl.pallas_call(
        _body,
        grid=(H,),
        in_specs=[
            pl.BlockSpec((1, IN), lambda h: (0, 0)),
            pl.BlockSpec((IN, IN), lambda h: (0, 0)),
            pl.BlockSpec((H, D), lambda h: (0, 0)),
            pl.BlockSpec((H, BD, D), lambda h: (0, 0, 0)),
            pl.BlockSpec((1, M, BD), lambda h: (h, 0, 0)),
        ],
        out_specs=pl.BlockSpec((1, 1, D), lambda h: (h, 0, 0)),
        out_shape=jax.ShapeDtypeStruct((H, 1, D), jnp.float32),
        scratch_shapes=[pltpu.VMEM((H, BD), jnp.float32)],
        compiler_params=pltpu.CompilerParams(
            dimension_semantics=("arbitrary",),
        ),
    )(x2, W_q, b_q2, bin_proj, weight_matrix)
    return out


def kernel(x, W_q, b_q, bin_proj, weight_matrix):
    out = _fused(x.reshape(1, IN), W_q, b_q.reshape(H, D), bin_proj,
                 weight_matrix)
    return out.reshape(H * D)
